# baseline (speedup 1.0000x reference)
import jax
import jax.numpy as jnp
from jax import lax
from jax.experimental import pallas as pl
from jax.experimental.pallas import tpu as pltpu
from jax.experimental.pallas import tpu_sc as plsc

_B = 16384
_D = 32
_NC = 2
_NS = 16
_NW = _NC * _NS
_BPW = _B // _NW
_CHUNK = 128
_NCHUNK = _BPW // _CHUNK
_LANES = 16
_G = _BPW // _LANES


def _gmf_body(uidx_hbm, iidx_hbm, utab_hbm, itab_hbm, wb_hbm, bias_hbm,
              out_hbm, uidx_v, iidx_v, urows_v, irows_v, wb_v, bias_v,
              out_v, sem):
    wid = lax.axis_index("s") * _NC + lax.axis_index("c")
    base = wid * _BPW
    crow = wid * _NCHUNK

    pltpu.sync_copy(uidx_hbm.at[pl.ds(crow, _NCHUNK)], uidx_v)
    pltpu.sync_copy(iidx_hbm.at[pl.ds(crow, _NCHUNK)], iidx_v)
    pltpu.sync_copy(wb_hbm, wb_v)
    pltpu.sync_copy(bias_hbm, bias_v)

    copies = []
    for j in range(_NCHUNK):
        copies.append(pltpu.async_copy(
            utab_hbm.at[uidx_v.at[j]],
            urows_v.at[pl.ds(j * _CHUNK, _CHUNK)], sem))
        copies.append(pltpu.async_copy(
            itab_hbm.at[iidx_v.at[j]],
            irows_v.at[pl.ds(j * _CHUNK, _CHUNK)], sem))
    for c in copies:
        c.wait()

    bias = bias_v[...]
    wcols = [wb_v[d, :] for d in range(_D)]
    riota = lax.iota(jnp.int32, _LANES)

    def group(g, carry):
        rid = riota + g * _LANES
        acc = bias
        for d in range(_D):
            cd = jnp.full((_LANES,), d, jnp.int32)
            u = plsc.load_gather(urows_v, [rid, cd])
            it = plsc.load_gather(irows_v, [rid, cd])
            acc = acc + u * it * wcols[d]
        out_v[pl.ds(g * _LANES, _LANES)] = acc
        return carry

    lax.fori_loop(0, _G, group, 0)

    pltpu.sync_copy(out_v, out_hbm.at[pl.ds(base, _BPW)])


def kernel(user_indices, item_indices, user_table, item_table, fc_w, fc_b):
    uidx = user_indices.astype(jnp.int32).reshape(_NW * _NCHUNK, _CHUNK)
    iidx = item_indices.astype(jnp.int32).reshape(_NW * _NCHUNK, _CHUNK)
    w = fc_w.reshape(_D).astype(jnp.float32)
    wb = jnp.broadcast_to(w[:, None], (_D, _LANES))
    bias = jnp.broadcast_to(fc_b.reshape(()), (_LANES,)).astype(jnp.float32)

    run = pl.kernel(
        _gmf_body,
        out_type=jax.ShapeDtypeStruct((_B,), jnp.float32),
        mesh=plsc.VectorSubcoreMesh(
            core_axis_name="c", subcore_axis_name="s",
            num_cores=_NC, num_subcores=_NS),
        compiler_params=pltpu.CompilerParams(
            needs_layout_passes=False, use_tc_tiling_on_sc=False),
        scratch_types=[
            pltpu.VMEM((_NCHUNK, _CHUNK), jnp.int32),
            pltpu.VMEM((_NCHUNK, _CHUNK), jnp.int32),
            pltpu.VMEM((_BPW, _D), jnp.float32),
            pltpu.VMEM((_BPW, _D), jnp.float32),
            pltpu.VMEM((_D, _LANES), jnp.float32),
            pltpu.VMEM((_LANES,), jnp.float32),
            pltpu.VMEM((_BPW,), jnp.float32),
            pltpu.SemaphoreType.DMA,
        ],
    )
    out = run(uidx, iidx, user_table, item_table, wb, bias)
    return out.reshape(_B, 1)
